# baseline (speedup 1.0000x reference)
import jax
import jax.numpy as jnp
from jax import lax
from jax.experimental import pallas as pl
from jax.experimental.pallas import tpu as pltpu
from jax.experimental.pallas import tpu_sc as plsc

_EPS = 1e-05
_NSEG = 64
_D = 256
_B = 80
_NB = 625
_NW = 32
_LANES = 16
_NCH = _D // _LANES
_NG = _B // _LANES


def _sc_body(feat_hbm, ids_hbm, sums_hbm, sqs_hbm, feat_v, idx_v,
             acc_s, acc_q):
    cid = lax.axis_index("c")
    sid = lax.axis_index("s")
    wid = sid * 2 + cid

    def zrow(r, c):
        for j in range(_NCH):
            z = jnp.zeros((_LANES,), jnp.float32)
            acc_s[r, pl.ds(j * _LANES, _LANES)] = z
            acc_q[r, pl.ds(j * _LANES, _LANES)] = z
        return c
    lax.fori_loop(0, _NSEG, zrow, 0)

    nblocks = jnp.where(wid < _NB - (_NB // _NW) * _NW, _NB // _NW + 1,
                        _NB // _NW)

    def step(t, c):
        base = (wid + t * _NW) * _B
        pltpu.sync_copy(feat_hbm.at[pl.ds(base, _B)], feat_v)
        pltpu.sync_copy(ids_hbm.at[pl.ds(base, _B)], idx_v)

        first = idx_v[pl.ds(0, _LANES)][0]
        last = idx_v[pl.ds(_B - _LANES, _LANES)][_LANES - 1]

        @pl.when(first == last)
        def _pure():
            for j in range(_NCH):
                sl = pl.ds(j * _LANES, _LANES)

                def row(r, carry):
                    runs, runq = carry
                    v = feat_v[r, sl]
                    return runs + v, runq + v * v

                z = jnp.zeros((_LANES,), jnp.float32)
                runs, runq = lax.fori_loop(0, _B, row, (z, z))
                acc_s[first, sl] = acc_s[first, sl] + runs
                acc_q[first, sl] = acc_q[first, sl] + runq

        @pl.when(first != last)
        def _mixed():
            def group(g, c):
                chunk = idx_v[pl.ds(g * _LANES, _LANES)]
                for i in range(_LANES):
                    s = chunk[i]
                    r = g * _LANES + i

                    def chunks(jj, cc):
                        sl = pl.ds(pl.multiple_of(jj * _LANES, _LANES),
                                   _LANES)
                        v = feat_v[r, sl]
                        acc_s[s, sl] = acc_s[s, sl] + v
                        acc_q[s, sl] = acc_q[s, sl] + v * v
                        return cc
                    lax.fori_loop(0, _NCH, chunks, 0)
                return c
            lax.fori_loop(0, _NG, group, 0)
        return c
    lax.fori_loop(0, nblocks, step, 0)

    pltpu.sync_copy(acc_s, sums_hbm.at[wid])
    pltpu.sync_copy(acc_q, sqs_hbm.at[wid])


def _fin_body(sums_ref, sqs_ref, out_ref):
    s = jnp.sum(sums_ref[...], axis=0)
    q = jnp.sum(sqs_ref[...], axis=0)
    out_ref[...] = jnp.sqrt(jax.nn.relu(q - s * s) + _EPS)


def kernel(feat, segment_ids):
    n, d = feat.shape
    assert n == _NB * _B and d == _D
    ids32 = segment_ids.astype(jnp.int32)

    sc = pl.kernel(
        _sc_body,
        out_type=(
            jax.ShapeDtypeStruct((_NW, _NSEG, _D), jnp.float32),
            jax.ShapeDtypeStruct((_NW, _NSEG, _D), jnp.float32),
        ),
        mesh=plsc.VectorSubcoreMesh(core_axis_name="c", subcore_axis_name="s"),
        scratch_types=[
            pltpu.VMEM((_B, _D), jnp.float32),
            pltpu.VMEM((_B,), jnp.int32),
            pltpu.VMEM((_NSEG, _D), jnp.float32),
            pltpu.VMEM((_NSEG, _D), jnp.float32),
        ],
    )
    sums, sqs = sc(feat, ids32)

    out = pl.pallas_call(
        _fin_body,
        out_shape=jax.ShapeDtypeStruct((_NSEG, _D), jnp.float32),
    )(sums, sqs)
    return out


# speedup vs baseline: 1.4991x; 1.4991x over previous
import jax
import jax.numpy as jnp
from jax import lax
from jax.experimental import pallas as pl
from jax.experimental.pallas import tpu as pltpu
from jax.experimental.pallas import tpu_sc as plsc

_EPS = 1e-05
_NSEG = 64
_D = 256
_B = 80
_NB = 625
_NW = 32
_LANES = 16
_NCH = _D // _LANES
_NG = _B // _LANES


def _sc_body(feat_hbm, ids_hbm, sums_hbm, sqs_hbm, feat_v, idx_v,
             acc_s, acc_q):
    cid = lax.axis_index("c")
    sid = lax.axis_index("s")
    wid = sid * 2 + cid

    def zrow(r, c):
        for j in range(_NCH):
            z = jnp.zeros((_LANES,), jnp.float32)
            acc_s[r, pl.ds(j * _LANES, _LANES)] = z
            acc_q[r, pl.ds(j * _LANES, _LANES)] = z
        return c
    lax.fori_loop(0, _NSEG, zrow, 0)

    nblocks = jnp.where(wid < _NB - (_NB // _NW) * _NW, _NB // _NW + 1,
                        _NB // _NW)

    def step(t, c):
        base = (wid + t * _NW) * _B
        pltpu.sync_copy(feat_hbm.at[pl.ds(base, _B)], feat_v)
        pltpu.sync_copy(ids_hbm.at[pl.ds(base, _B)], idx_v)

        first = idx_v[pl.ds(0, _LANES)][0]
        last = idx_v[pl.ds(_B - _LANES, _LANES)][_LANES - 1]

        @pl.when(first == last)
        def _pure():
            for j in range(_NCH):
                sl = pl.ds(j * _LANES, _LANES)

                def group(g, carry):
                    runs0, runq0, runs1, runq1 = carry
                    base_r = g * _LANES
                    for i in range(0, _LANES, 2):
                        v0 = feat_v[base_r + i, sl]
                        v1 = feat_v[base_r + i + 1, sl]
                        runs0 = runs0 + v0
                        runq0 = runq0 + v0 * v0
                        runs1 = runs1 + v1
                        runq1 = runq1 + v1 * v1
                    return runs0, runq0, runs1, runq1

                z = jnp.zeros((_LANES,), jnp.float32)
                runs0, runq0, runs1, runq1 = lax.fori_loop(
                    0, _NG, group, (z, z, z, z))
                acc_s[first, sl] = acc_s[first, sl] + (runs0 + runs1)
                acc_q[first, sl] = acc_q[first, sl] + (runq0 + runq1)

        @pl.when(first != last)
        def _mixed():
            def group(g, c):
                chunk = idx_v[pl.ds(g * _LANES, _LANES)]
                for i in range(_LANES):
                    s = chunk[i]
                    r = g * _LANES + i

                    def chunks(jj, cc):
                        sl = pl.ds(pl.multiple_of(jj * _LANES, _LANES),
                                   _LANES)
                        v = feat_v[r, sl]
                        acc_s[s, sl] = acc_s[s, sl] + v
                        acc_q[s, sl] = acc_q[s, sl] + v * v
                        return cc
                    lax.fori_loop(0, _NCH, chunks, 0)
                return c
            lax.fori_loop(0, _NG, group, 0)
        return c
    lax.fori_loop(0, nblocks, step, 0)

    pltpu.sync_copy(acc_s, sums_hbm.at[wid])
    pltpu.sync_copy(acc_q, sqs_hbm.at[wid])


def _fin_body(sums_ref, sqs_ref, out_ref):
    s = jnp.sum(sums_ref[...], axis=0)
    q = jnp.sum(sqs_ref[...], axis=0)
    out_ref[...] = jnp.sqrt(jax.nn.relu(q - s * s) + _EPS)


def kernel(feat, segment_ids):
    n, d = feat.shape
    assert n == _NB * _B and d == _D
    ids32 = segment_ids.astype(jnp.int32)

    sc = pl.kernel(
        _sc_body,
        out_type=(
            jax.ShapeDtypeStruct((_NW, _NSEG, _D), jnp.float32),
            jax.ShapeDtypeStruct((_NW, _NSEG, _D), jnp.float32),
        ),
        mesh=plsc.VectorSubcoreMesh(core_axis_name="c", subcore_axis_name="s"),
        scratch_types=[
            pltpu.VMEM((_B, _D), jnp.float32),
            pltpu.VMEM((_B,), jnp.int32),
            pltpu.VMEM((_NSEG, _D), jnp.float32),
            pltpu.VMEM((_NSEG, _D), jnp.float32),
        ],
    )
    sums, sqs = sc(feat, ids32)

    out = pl.pallas_call(
        _fin_body,
        out_shape=jax.ShapeDtypeStruct((_NSEG, _D), jnp.float32),
    )(sums, sqs)
    return out


# speedup vs baseline: 3.1233x; 2.0835x over previous
import jax
import jax.numpy as jnp
from jax import lax
from jax.experimental import pallas as pl
from jax.experimental.pallas import tpu as pltpu
from jax.experimental.pallas import tpu_sc as plsc

_EPS = 1e-05
_NSEG = 64
_D = 256
_N = 50000
_LANES = 16
_NCH = _D // _LANES

_B = 80
_NG = _B // _LANES
_NW = 32
_NBW = 5
_RPW = _NBW * _B
_R_TC = _N - _NW * _RPW
_TC_BLOCK = 1200
_TC_STEPS = _R_TC // _TC_BLOCK


def _compute_block(feat_v, idx_all, acc_s, acc_q, t):
    tb = t * _B
    first = idx_all[pl.ds(tb, _LANES)][0]
    last = idx_all[pl.ds(tb + _B - _LANES, _LANES)][_LANES - 1]

    @pl.when(first == last)
    def _pure():
        for j in range(_NCH):
            sl = pl.ds(j * _LANES, _LANES)

            def group(g, carry):
                runs0, runq0, runs1, runq1 = carry
                base_r = g * _LANES
                for i in range(0, _LANES, 2):
                    v0 = feat_v[base_r + i, sl]
                    v1 = feat_v[base_r + i + 1, sl]
                    runs0 = runs0 + v0
                    runq0 = runq0 + v0 * v0
                    runs1 = runs1 + v1
                    runq1 = runq1 + v1 * v1
                return runs0, runq0, runs1, runq1

            z = jnp.zeros((_LANES,), jnp.float32)
            runs0, runq0, runs1, runq1 = lax.fori_loop(
                0, _NG, group, (z, z, z, z))
            acc_s[first, sl] = acc_s[first, sl] + (runs0 + runs1)
            acc_q[first, sl] = acc_q[first, sl] + (runq0 + runq1)

    @pl.when(first != last)
    def _mixed():
        def group(g, c):
            chunk = idx_all[pl.ds(tb + g * _LANES, _LANES)]
            for i in range(_LANES):
                s = chunk[i]
                r = g * _LANES + i

                def chunks(jj, cc):
                    sl = pl.ds(pl.multiple_of(jj * _LANES, _LANES), _LANES)
                    v = feat_v[r, sl]
                    acc_s[s, sl] = acc_s[s, sl] + v
                    acc_q[s, sl] = acc_q[s, sl] + v * v
                    return cc
                lax.fori_loop(0, _NCH, chunks, 0)
            return c
        lax.fori_loop(0, _NG, group, 0)


def _sc_body(feat_hbm, ids_hbm, sums_hbm, sqs_hbm,
             feat_v0, feat_v1, idx_all, acc_s, acc_q, sem0, sem1):
    cid = lax.axis_index("c")
    sid = lax.axis_index("s")
    wid = sid * 2 + cid
    row0 = _R_TC + wid * _RPW

    pltpu.sync_copy(ids_hbm.at[pl.ds(row0, _RPW)], idx_all)

    def zrow(r, c):
        for j in range(_NCH):
            z = jnp.zeros((_LANES,), jnp.float32)
            acc_s[r, pl.ds(j * _LANES, _LANES)] = z
            acc_q[r, pl.ds(j * _LANES, _LANES)] = z
        return c
    lax.fori_loop(0, _NSEG, zrow, 0)

    bufs = (feat_v0, feat_v1)
    sems = (sem0, sem1)

    def dma(t, p):
        return pltpu.make_async_copy(
            feat_hbm.at[pl.ds(row0 + t * _B, _B)], bufs[p], sems[p])

    dma(0, 0).start()
    dma(1, 1).start()

    def pair(k, c):
        for p in range(2):
            b = 2 * k + p

            @pl.when(b < _NBW)
            def _():
                dma(b, p).wait()
                _compute_block(bufs[p], idx_all, acc_s, acc_q, b)

                @pl.when(b + 2 < _NBW)
                def _():
                    dma(b + 2, p).start()
        return c
    lax.fori_loop(0, (_NBW + 1) // 2, pair, 0)

    pltpu.sync_copy(acc_s, sums_hbm.at[wid])
    pltpu.sync_copy(acc_q, sqs_hbm.at[wid])


def _tc_body(ids_ref, feat_ref, s_out, q_out, acc_s, acc_q):
    i = pl.program_id(0)

    @pl.when(i == 0)
    def _():
        acc_s[...] = jnp.zeros_like(acc_s)
        acc_q[...] = jnp.zeros_like(acc_q)

    x = feat_ref[...]
    ids = ids_ref[0, 0, :]
    seg_iota = lax.broadcasted_iota(jnp.int32, (_NSEG, _TC_BLOCK), 0)
    oh = (seg_iota == ids[None, :]).astype(jnp.bfloat16)
    xh = x.astype(jnp.bfloat16)
    xl = (x - xh.astype(jnp.float32)).astype(jnp.bfloat16)
    xq = (x * x).astype(jnp.bfloat16)
    acc_s[...] += (
        jax.lax.dot(oh, xh, preferred_element_type=jnp.float32)
        + jax.lax.dot(oh, xl, preferred_element_type=jnp.float32))
    acc_q[...] += jax.lax.dot(oh, xq, preferred_element_type=jnp.float32)

    @pl.when(i == _TC_STEPS - 1)
    def _():
        s_out[...] = acc_s[...]
        q_out[...] = acc_q[...]


def _fin_body(tc_s_ref, tc_q_ref, sums_ref, sqs_ref, out_ref):
    s = tc_s_ref[...] + jnp.sum(sums_ref[...], axis=0)
    q = tc_q_ref[...] + jnp.sum(sqs_ref[...], axis=0)
    out_ref[...] = jnp.sqrt(jax.nn.relu(q - s * s) + _EPS)


def kernel(feat, segment_ids):
    n, d = feat.shape
    assert n == _N and d == _D
    ids32 = segment_ids.astype(jnp.int32)
    ids_tc = ids32[:_R_TC].reshape(_TC_STEPS, 1, _TC_BLOCK)

    sums, sqs = pl.kernel(
        _sc_body,
        out_type=(
            jax.ShapeDtypeStruct((_NW, _NSEG, _D), jnp.float32),
            jax.ShapeDtypeStruct((_NW, _NSEG, _D), jnp.float32),
        ),
        mesh=plsc.VectorSubcoreMesh(core_axis_name="c", subcore_axis_name="s"),
        scratch_types=[
            pltpu.VMEM((_B, _D), jnp.float32),
            pltpu.VMEM((_B, _D), jnp.float32),
            pltpu.VMEM((_RPW,), jnp.int32),
            pltpu.VMEM((_NSEG, _D), jnp.float32),
            pltpu.VMEM((_NSEG, _D), jnp.float32),
            pltpu.SemaphoreType.DMA,
            pltpu.SemaphoreType.DMA,
        ],
    )(feat, ids32)

    tc_s, tc_q = pl.pallas_call(
        _tc_body,
        grid=(_TC_STEPS,),
        in_specs=[
            pl.BlockSpec((1, 1, _TC_BLOCK), lambda i: (i, 0, 0)),
            pl.BlockSpec((_TC_BLOCK, d), lambda i: (i, 0)),
        ],
        out_specs=[
            pl.BlockSpec((_NSEG, d), lambda i: (0, 0)),
            pl.BlockSpec((_NSEG, d), lambda i: (0, 0)),
        ],
        out_shape=[
            jax.ShapeDtypeStruct((_NSEG, d), jnp.float32),
            jax.ShapeDtypeStruct((_NSEG, d), jnp.float32),
        ],
        scratch_shapes=[
            pltpu.VMEM((_NSEG, d), jnp.float32),
            pltpu.VMEM((_NSEG, d), jnp.float32),
        ],
    )(ids_tc, feat)

    out = pl.pallas_call(
        _fin_body,
        out_shape=jax.ShapeDtypeStruct((_NSEG, d), jnp.float32),
    )(tc_s, tc_q, sums, sqs)
    return out
